# baseline (speedup 1.0000x reference)
import functools

import jax
import jax.numpy as jnp
from jax import lax
from jax.experimental import pallas as pl
from jax.experimental.pallas import tpu as pltpu
from jax.experimental.pallas import tpu_sc as plsc

D = 64
NUM_PROMPTS = 4

_info = plsc.get_sparse_core_info()
NC, NS, LANES = _info.num_cores, _info.num_subcores, _info.num_lanes
NW = NC * NS

CHUNK = 128


def _make_gather(n):
    assert n % (NW * CHUNK) == 0
    per_w = n // NW
    n_chunks = per_w // CHUNK
    mesh = plsc.VectorSubcoreMesh(core_axis_name="c", subcore_axis_name="s")

    @functools.partial(
        pl.kernel,
        mesh=mesh,
        out_type=jax.ShapeDtypeStruct((n, D), jnp.float32),
        compiler_params=pltpu.CompilerParams(
            needs_layout_passes=False, use_tc_tiling_on_sc=False),
        scratch_types=[
            pltpu.VMEM((CHUNK,), jnp.int32),
            pltpu.VMEM((CHUNK, D), jnp.float32),
            pltpu.VMEM((NUM_PROMPTS, D), jnp.float32),
            pltpu.VMEM((LANES,), jnp.int32),
            pltpu.SemaphoreType.DMA,
        ],
    )
    def k(idx_hbm, table_hbm, prompt_hbm, out_hbm, idx_v, rows_v, prompt_v,
          red_v, sem):
        wid = lax.axis_index("s") * NC + lax.axis_index("c")
        base = wid * per_w
        pltpu.sync_copy(prompt_hbm, prompt_v)

        zeros = jnp.zeros((LANES,), jnp.int32)
        ones = jnp.ones((LANES,), jnp.int32)

        def chunk_body(g, carry):
            off = base + g * CHUNK
            pltpu.sync_copy(idx_hbm.at[pl.ds(off, CHUNK)], idx_v)
            pltpu.async_copy(table_hbm.at[idx_v], rows_v, sem).wait()

            def acc_group(i, acc):
                v = idx_v[pl.ds(i * LANES, LANES)]
                hit = (v >= 1) & (v <= NUM_PROMPTS)
                return acc | jnp.where(hit, ones, zeros)

            acc = lax.fori_loop(0, CHUNK // LANES, acc_group, zeros)
            lane = lax.iota(jnp.int32, LANES)
            for sh in (8, 4, 2, 1):
                red_v[...] = acc
                acc = acc | plsc.load_gather(red_v, [(lane + sh) & (LANES - 1)])
            any_hit = acc[0]

            @pl.when(any_hit > 0)
            def _():
                def fix_group(i, c2):
                    v = idx_v[pl.ds(i * LANES, LANES)]
                    hit = (v >= 1) & (v <= NUM_PROMPTS)
                    rows = lax.iota(jnp.int32, LANES) + i * LANES
                    pid = jnp.where(hit, v - 1, zeros)
                    for c in range(D):
                        cols = jnp.full((LANES,), c, jnp.int32)
                        vals = plsc.load_gather(prompt_v, [pid, cols])
                        plsc.store_scatter(rows_v, [rows, cols], vals,
                                           mask=hit)
                    return c2

                lax.fori_loop(0, CHUNK // LANES, fix_group, 0)

            pltpu.sync_copy(rows_v, out_hbm.at[pl.ds(off, CHUNK)])
            return carry

        lax.fori_loop(0, n_chunks, chunk_body, 0)

    return k


def kernel(input, table, prompt_params):
    b, l = input.shape
    n = b * l
    idx_flat = input.reshape(n)
    out = _make_gather(n)(idx_flat, table, prompt_params)
    return out.reshape(b, l, D)


# speedup vs baseline: 1.1842x; 1.1842x over previous
import functools

import jax
import jax.numpy as jnp
from jax import lax
from jax.experimental import pallas as pl
from jax.experimental.pallas import tpu as pltpu
from jax.experimental.pallas import tpu_sc as plsc

D = 64
DPAD = 128
NUM_PROMPTS = 4

_info = plsc.get_sparse_core_info()
NC, NS, LANES = _info.num_cores, _info.num_subcores, _info.num_lanes
NW = NC * NS

GATHER = 128
NGATHER = 5
CHUNK = GATHER * NGATHER


def _make_gather(n):
    assert n % (NW * CHUNK) == 0
    per_w = n // NW
    n_chunks = per_w // CHUNK
    assert n_chunks % 2 == 0
    mesh = plsc.VectorSubcoreMesh(core_axis_name="c", subcore_axis_name="s")

    @functools.partial(
        pl.kernel,
        mesh=mesh,
        out_type=jax.ShapeDtypeStruct((n, DPAD), jnp.float32),
        compiler_params=pltpu.CompilerParams(
            needs_layout_passes=False, use_tc_tiling_on_sc=False),
        scratch_types=[
            pltpu.VMEM((2, NGATHER, GATHER), jnp.int32),
            pltpu.VMEM((2, CHUNK, D), jnp.float32),
            pltpu.VMEM((NUM_PROMPTS, D), jnp.float32),
            pltpu.VMEM((LANES,), jnp.int32),
            pltpu.SemaphoreType.DMA((2,)),
            pltpu.SemaphoreType.DMA((2,)),
        ],
    )
    def k(idx_hbm, table_hbm, prompt_hbm, out_hbm, idx_v, rows_v, prompt_v,
          red_v, gsem, wsem):
        wid = lax.axis_index("s") * NC + lax.axis_index("c")
        base_row = wid * (per_w // GATHER)
        base = wid * per_w
        pltpu.sync_copy(prompt_hbm, prompt_v)

        zeros = jnp.zeros((LANES,), jnp.int32)
        ones = jnp.ones((LANES,), jnp.int32)
        lane = lax.iota(jnp.int32, LANES)

        def load_and_gather(g, b):
            pltpu.sync_copy(
                idx_hbm.at[pl.ds(base_row + g * NGATHER, NGATHER)],
                idx_v.at[b])
            for j in range(NGATHER):
                pltpu.async_copy(
                    table_hbm.at[idx_v.at[b, j]],
                    rows_v.at[b, pl.ds(j * GATHER, GATHER)],
                    gsem.at[b])

        def drain_gather(b):
            for j in range(NGATHER):
                pltpu.make_async_copy(
                    table_hbm.at[idx_v.at[b, j]],
                    rows_v.at[b, pl.ds(j * GATHER, GATHER)],
                    gsem.at[b]).wait()

        def or_reduce(acc):
            out = acc
            for sh in (8, 4, 2, 1):
                red_v[...] = out
                rot = plsc.load_gather(red_v, [(lane + sh) & (LANES - 1)])
                out = out | rot
            return out[0]

        def fix_chunk(b):
            acc = zeros
            for j in range(NGATHER):
                def acc_group(i, a, j=j):
                    v = idx_v[b, j, pl.ds(i * LANES, LANES)]
                    hit = (v >= 1) & (v <= NUM_PROMPTS)
                    return a | jnp.where(hit, ones, zeros)

                acc = lax.fori_loop(0, GATHER // LANES, acc_group, acc)
            any_hit = or_reduce(acc)

            @pl.when(any_hit > 0)
            def _():
                for j in range(NGATHER):
                    def fix_group(i, c2, j=j):
                        v = idx_v[b, j, pl.ds(i * LANES, LANES)]
                        hit = (v >= 1) & (v <= NUM_PROMPTS)
                        rows = lane + (j * (GATHER // LANES) + i) * LANES
                        pid = jnp.where(hit, v - 1, zeros)
                        for c in range(D):
                            cols = jnp.full((LANES,), c, jnp.int32)
                            vals = plsc.load_gather(prompt_v, [pid, cols])
                            plsc.store_scatter(rows_v.at[b], [rows, cols],
                                               vals, mask=hit)
                        return c2

                    lax.fori_loop(0, GATHER // LANES, fix_group, 0)

        def write_chunk(b, off):
            pltpu.async_copy(
                rows_v.at[b],
                out_hbm.at[pl.ds(off, CHUNK), pl.ds(0, D)],
                wsem.at[b])

        def drain_write(b, off):
            pltpu.make_async_copy(
                rows_v.at[b],
                out_hbm.at[pl.ds(off, CHUNK), pl.ds(0, D)],
                wsem.at[b]).wait()

        load_and_gather(0, 0)

        def pair_body(g2, carry):
            for t in (0, 1):
                g = g2 * 2 + t

                @pl.when(g + 1 < n_chunks)
                def _(g=g, t=t):
                    @pl.when(g >= 1)
                    def _():
                        drain_write(1 - t, base + (g - 1) * CHUNK)

                    load_and_gather(g + 1, 1 - t)

                drain_gather(t)
                fix_chunk(t)
                write_chunk(t, base + g * CHUNK)
            return carry

        lax.fori_loop(0, n_chunks // 2, pair_body, 0)
        drain_write(0, base + (n_chunks - 2) * CHUNK)
        drain_write(1, base + (n_chunks - 1) * CHUNK)

    return k


def kernel(input, table, prompt_params):
    b, l = input.shape
    n = b * l
    idx2d = input.reshape(n // GATHER, GATHER)
    out = _make_gather(n)(idx2d, table, prompt_params)
    return out[:, :D].reshape(b, l, D)


# speedup vs baseline: 1.2262x; 1.0354x over previous
import functools

import jax
import jax.numpy as jnp
from jax import lax
from jax.experimental import pallas as pl
from jax.experimental.pallas import tpu as pltpu
from jax.experimental.pallas import tpu_sc as plsc

D = 64
DPAD = 128
NUM_PROMPTS = 4

_info = plsc.get_sparse_core_info()
NC, NS, LANES = _info.num_cores, _info.num_subcores, _info.num_lanes
NW = NC * NS

GATHER = 128
NGATHER = 1
CHUNK = GATHER * NGATHER


def _make_gather(n):
    assert n % (NW * CHUNK) == 0
    per_w = n // NW
    n_chunks = per_w // CHUNK
    assert n_chunks % 2 == 0
    mesh = plsc.VectorSubcoreMesh(core_axis_name="c", subcore_axis_name="s")

    @functools.partial(
        pl.kernel,
        mesh=mesh,
        out_type=jax.ShapeDtypeStruct((n, DPAD), jnp.float32),
        compiler_params=pltpu.CompilerParams(
            needs_layout_passes=False, use_tc_tiling_on_sc=False),
        scratch_types=[
            pltpu.VMEM((2, NGATHER, GATHER), jnp.int32),
            pltpu.VMEM((2, CHUNK, DPAD), jnp.float32),
            pltpu.VMEM((NUM_PROMPTS, D), jnp.float32),
            pltpu.VMEM((LANES,), jnp.int32),
            pltpu.SemaphoreType.DMA((2,)),
            pltpu.SemaphoreType.DMA((2,)),
        ],
    )
    def k(idx_hbm, table_hbm, prompt_hbm, out_hbm, idx_v, rows_v, prompt_v,
          red_v, gsem, wsem):
        wid = lax.axis_index("s") * NC + lax.axis_index("c")
        base_row = wid * (per_w // GATHER)
        base = wid * per_w
        pltpu.sync_copy(prompt_hbm, prompt_v)

        zeros = jnp.zeros((LANES,), jnp.int32)
        ones = jnp.ones((LANES,), jnp.int32)
        lane = lax.iota(jnp.int32, LANES)

        def load_and_gather(g, b):
            pltpu.sync_copy(
                idx_hbm.at[pl.ds(base_row + g * NGATHER, NGATHER)],
                idx_v.at[b])
            for j in range(NGATHER):
                pltpu.async_copy(
                    table_hbm.at[idx_v.at[b, j]],
                    rows_v.at[b, pl.ds(j * GATHER, GATHER)],
                    gsem.at[b])

        def drain_gather(b):
            for j in range(NGATHER):
                pltpu.make_async_copy(
                    table_hbm.at[idx_v.at[b, j]],
                    rows_v.at[b, pl.ds(j * GATHER, GATHER)],
                    gsem.at[b]).wait()

        def or_reduce(acc):
            out = acc
            for sh in (8, 4, 2, 1):
                red_v[...] = out
                rot = plsc.load_gather(red_v, [(lane + sh) & (LANES - 1)])
                out = out | rot
            return out[0]

        def fix_chunk(b):
            acc = zeros
            for j in range(NGATHER):
                def acc_group(i, a, j=j):
                    v = idx_v[b, j, pl.ds(i * LANES, LANES)]
                    hit = (v >= 1) & (v <= NUM_PROMPTS)
                    return a | jnp.where(hit, ones, zeros)

                acc = lax.fori_loop(0, GATHER // LANES, acc_group, acc)
            any_hit = or_reduce(acc)

            @pl.when(any_hit > 0)
            def _():
                for j in range(NGATHER):
                    def fix_group(i, c2, j=j):
                        v = idx_v[b, j, pl.ds(i * LANES, LANES)]
                        hit = (v >= 1) & (v <= NUM_PROMPTS)
                        rows = lane + (j * (GATHER // LANES) + i) * LANES
                        pid = jnp.where(hit, v - 1, zeros)
                        for c in range(D):
                            cols = jnp.full((LANES,), c, jnp.int32)
                            vals = plsc.load_gather(prompt_v, [pid, cols])
                            plsc.store_scatter(rows_v.at[b], [rows, cols],
                                               vals, mask=hit)
                        return c2

                    lax.fori_loop(0, GATHER // LANES, fix_group, 0)

        def write_chunk(b, off):
            pltpu.async_copy(
                rows_v.at[b],
                out_hbm.at[pl.ds(off, CHUNK)],
                wsem.at[b])

        def drain_write(b, off):
            pltpu.make_async_copy(
                rows_v.at[b],
                out_hbm.at[pl.ds(off, CHUNK)],
                wsem.at[b]).wait()

        load_and_gather(0, 0)

        def pair_body(g2, carry):
            for t in (0, 1):
                g = g2 * 2 + t

                @pl.when(g + 1 < n_chunks)
                def _(g=g, t=t):
                    @pl.when(g >= 1)
                    def _():
                        drain_write(1 - t, base + (g - 1) * CHUNK)

                    load_and_gather(g + 1, 1 - t)

                drain_gather(t)
                fix_chunk(t)
                write_chunk(t, base + g * CHUNK)
            return carry

        lax.fori_loop(0, n_chunks // 2, pair_body, 0)
        drain_write(0, base + (n_chunks - 2) * CHUNK)
        drain_write(1, base + (n_chunks - 1) * CHUNK)

    return k


def kernel(input, table, prompt_params):
    b, l = input.shape
    n = b * l
    idx2d = input.reshape(n // GATHER, GATHER)
    table128 = jnp.pad(table, ((0, 0), (0, DPAD - D)))
    out = _make_gather(n)(idx2d, table128, prompt_params)
    return out[:, :D].reshape(b, l, D)


# speedup vs baseline: 1.2449x; 1.0153x over previous
import functools

import jax
import jax.numpy as jnp
from jax import lax
from jax.experimental import pallas as pl
from jax.experimental.pallas import tpu as pltpu
from jax.experimental.pallas import tpu_sc as plsc

D = 64
DPAD = 128
NUM_PROMPTS = 4

_info = plsc.get_sparse_core_info()
NC, NS, LANES = _info.num_cores, _info.num_subcores, _info.num_lanes
NW = NC * NS

GATHER = 128
NGATHER = 1
CHUNK = GATHER * NGATHER
NBUF = 5


def _make_gather(n):
    assert n % (NW * CHUNK) == 0
    per_w = n // NW
    n_chunks = per_w // CHUNK
    assert n_chunks % NBUF == 0
    mesh = plsc.VectorSubcoreMesh(core_axis_name="c", subcore_axis_name="s")

    @functools.partial(
        pl.kernel,
        mesh=mesh,
        out_type=jax.ShapeDtypeStruct((n, DPAD), jnp.float32),
        compiler_params=pltpu.CompilerParams(
            needs_layout_passes=False, use_tc_tiling_on_sc=False),
        scratch_types=[
            pltpu.VMEM((NBUF, NGATHER, GATHER), jnp.int32),
            pltpu.VMEM((NBUF, CHUNK, DPAD), jnp.float32),
            pltpu.VMEM((NUM_PROMPTS, D), jnp.float32),
            pltpu.VMEM((LANES,), jnp.int32),
            pltpu.SemaphoreType.DMA((NBUF,)),
            pltpu.SemaphoreType.DMA((NBUF,)),
        ],
    )
    def k(idx_hbm, table_hbm, prompt_hbm, out_hbm, idx_v, rows_v, prompt_v,
          red_v, gsem, wsem):
        wid = lax.axis_index("s") * NC + lax.axis_index("c")
        base_row = wid * (per_w // GATHER)
        base = wid * per_w
        pltpu.sync_copy(prompt_hbm, prompt_v)

        zeros = jnp.zeros((LANES,), jnp.int32)
        ones = jnp.ones((LANES,), jnp.int32)
        lane = lax.iota(jnp.int32, LANES)

        def load_and_gather(g, b):
            pltpu.sync_copy(
                idx_hbm.at[pl.ds(base_row + g * NGATHER, NGATHER)],
                idx_v.at[b])
            for j in range(NGATHER):
                pltpu.async_copy(
                    table_hbm.at[idx_v.at[b, j]],
                    rows_v.at[b, pl.ds(j * GATHER, GATHER)],
                    gsem.at[b])

        def drain_gather(b):
            for j in range(NGATHER):
                pltpu.make_async_copy(
                    table_hbm.at[idx_v.at[b, j]],
                    rows_v.at[b, pl.ds(j * GATHER, GATHER)],
                    gsem.at[b]).wait()

        def or_reduce(acc):
            out = acc
            for sh in (8, 4, 2, 1):
                red_v[...] = out
                rot = plsc.load_gather(red_v, [(lane + sh) & (LANES - 1)])
                out = out | rot
            return out[0]

        def fix_chunk(b):
            acc = zeros
            for j in range(NGATHER):
                def acc_group(i, a, j=j):
                    v = idx_v[b, j, pl.ds(i * LANES, LANES)]
                    hit = (v >= 1) & (v <= NUM_PROMPTS)
                    return a | jnp.where(hit, ones, zeros)

                acc = lax.fori_loop(0, GATHER // LANES, acc_group, acc)
            any_hit = or_reduce(acc)

            @pl.when(any_hit > 0)
            def _():
                for j in range(NGATHER):
                    def fix_group(i, c2, j=j):
                        v = idx_v[b, j, pl.ds(i * LANES, LANES)]
                        hit = (v >= 1) & (v <= NUM_PROMPTS)
                        rows = lane + (j * (GATHER // LANES) + i) * LANES
                        pid = jnp.where(hit, v - 1, zeros)
                        for c in range(D):
                            cols = jnp.full((LANES,), c, jnp.int32)
                            vals = plsc.load_gather(prompt_v, [pid, cols])
                            plsc.store_scatter(rows_v.at[b], [rows, cols],
                                               vals, mask=hit)
                        return c2

                    lax.fori_loop(0, GATHER // LANES, fix_group, 0)

        def write_chunk(b, off):
            pltpu.async_copy(
                rows_v.at[b],
                out_hbm.at[pl.ds(off, CHUNK)],
                wsem.at[b])

        def drain_write(b, off):
            pltpu.make_async_copy(
                rows_v.at[b],
                out_hbm.at[pl.ds(off, CHUNK)],
                wsem.at[b]).wait()

        for t in range(NBUF - 1):
            load_and_gather(t, t)

        def group_body(gq, carry):
            for t in range(NBUF):
                g = gq * NBUF + t
                pb = (t - 1) % NBUF

                @pl.when(g >= 1)
                def _(g=g, pb=pb):
                    drain_write(pb, base + (g - 1) * CHUNK)

                @pl.when(g + NBUF - 1 < n_chunks)
                def _(g=g, pb=pb):
                    load_and_gather(g + NBUF - 1, pb)

                drain_gather(t)
                fix_chunk(t)
                write_chunk(t, base + g * CHUNK)
            return carry

        lax.fori_loop(0, n_chunks // NBUF, group_body, 0)
        drain_write((n_chunks - 1) % NBUF, base + (n_chunks - 1) * CHUNK)

    return k


def kernel(input, table, prompt_params):
    b, l = input.shape
    n = b * l
    idx2d = input.reshape(n // GATHER, GATHER)
    table128 = jnp.pad(table, ((0, 0), (0, DPAD - D)))
    out = _make_gather(n)(idx2d, table128, prompt_params)
    return out[:, :D].reshape(b, l, D)
